# baseline (device time: 75729 ns/iter reference)
import jax
import jax.numpy as jnp
from jax import lax
from jax.experimental import pallas as pl
from jax.experimental.pallas import tpu as pltpu

B, S, H, Dh, Dr = 2, 512, 16, 128, 32
D = 2048
DC_SH = 128
HG = 4
NHG = H // HG
NB = 8
WQB = D // NB
N_ATTN = B * NHG
F32 = jnp.float32
BF16 = jnp.bfloat16
SCALE = (Dh + Dr) ** -0.5


def _dot(a, b):
    return jnp.dot(a, b, preferred_element_type=F32)


def _mega_body(x_ref, wdkv_ref, wuk_ref, wuv_ref, wkr_ref, wq_ref, wqr_ref,
               wo_ref,
               out_ref,
               q_s, k_s, v_s, qr_s, kr_s,
               c_ref, wuk_bf, wuv_bf, peer_c, peer_wuk, peer_wuv,
               send_sems, recv_sems):
    g = pl.program_id(0)
    my_x = lax.axis_index("x")
    my_y = lax.axis_index("y")
    my_z = lax.axis_index("z")
    peer = (my_x, 1 - my_y, my_z)

    def exchange_rdmas():
        return [
            pltpu.make_async_remote_copy(
                src_ref=src, dst_ref=dst,
                send_sem=send_sems.at[i], recv_sem=recv_sems.at[i],
                device_id=peer, device_id_type=pl.DeviceIdType.MESH)
            for i, (src, dst) in enumerate(((c_ref, peer_c),
                                            (wuk_bf, peer_wuk),
                                            (wuv_bf, peer_wuv)))
        ]

    @pl.when(g == 0)
    def _():
        barrier_sem = pltpu.get_barrier_semaphore()
        pl.semaphore_signal(barrier_sem, inc=1, device_id=peer,
                            device_id_type=pl.DeviceIdType.MESH)
        pl.semaphore_wait(barrier_sem, 1)

        wuk_bf[...] = wuk_ref[...].astype(BF16)
        wuv_bf[...] = wuv_ref[...].astype(BF16)
        c_ref[...] = _dot(x_ref[...], wdkv_ref[...]).astype(BF16)

        for rdma in exchange_rdmas():
            rdma.start()

        kr_s[...] = _dot(x_ref[...], wkr_ref[...]).astype(BF16)
        qr_s[...] = (_dot(x_ref[...], wqr_ref[...]) * SCALE).astype(BF16)
        k_s[...] = _dot(c_ref[...], wuk_bf[...]).astype(BF16)
        v_s[...] = _dot(c_ref[...], wuv_bf[...]).astype(BF16)

    @pl.when(g < NB)
    def _():
        q_s[:, pl.ds(g * WQB, WQB)] = (
            _dot(x_ref[...], wq_ref[...]) * SCALE).astype(BF16)

    @pl.when(g == NB - 1)
    def _():
        for rdma in exchange_rdmas():
            rdma.wait()
        k_s[...] = (k_s[...] + _dot(peer_c[...], peer_wuk[...])).astype(BF16)
        v_s[...] = (v_s[...] + _dot(peer_c[...], peer_wuv[...])).astype(BF16)

    @pl.when(g >= NB)
    def _():
        j = g - NB
        b = j // NHG
        hg = j % NHG
        row = pl.ds(b * S, S)
        kr = kr_s[row, :]
        qr_grp = qr_s[row, pl.ds(hg * HG * Dr, HG * Dr)]
        contract_last = (((1,), (1,)), ((), ()))
        o_heads = []
        for i in range(HG):
            col = pl.ds(hg * HG * Dh + i * Dh, Dh)
            q = q_s[row, col]
            qr = qr_grp[:, i * Dr:(i + 1) * Dr]
            k = k_s[row, col]
            v = v_s[row, col]
            scores = (lax.dot_general(q, k, contract_last,
                                      preferred_element_type=F32)
                      + lax.dot_general(qr, kr, contract_last,
                                        preferred_element_type=F32))
            p = jnp.exp(scores)
            recip = 1.0 / jnp.sum(p, axis=-1, keepdims=True)
            o = _dot(p.astype(BF16), v)
            o_heads.append((o * recip).astype(BF16))
        o4 = jnp.concatenate(o_heads, axis=1)
        partial = _dot(o4, wo_ref[...])

        @pl.when(hg == 0)
        def _():
            out_ref[...] = partial

        @pl.when(hg != 0)
        def _():
            out_ref[...] += partial


def kernel(x, Wdkv, Wuk, Wuv, Wq, Wqr, Wkr, Wo):
    x2 = x.reshape(B * S, D)

    out = pl.pallas_call(
        _mega_body,
        grid=(NB + N_ATTN,),
        out_shape=jax.ShapeDtypeStruct((B * S, D), F32),
        in_specs=[
            pl.BlockSpec((B * S, D), lambda g: (0, 0)),
            pl.BlockSpec((D, DC_SH), lambda g: (0, 0)),
            pl.BlockSpec((DC_SH, H * Dh), lambda g: (0, 0)),
            pl.BlockSpec((DC_SH, H * Dh), lambda g: (0, 0)),
            pl.BlockSpec((D, Dr), lambda g: (0, 0)),
            pl.BlockSpec((D, WQB),
                         lambda g: (0, jnp.minimum(g, NB - 1))),
            pl.BlockSpec((D, H * Dr), lambda g: (0, 0)),
            pl.BlockSpec((HG * Dh, D),
                         lambda g: (jnp.where(g < NB, 0,
                                              (g - NB) % NHG), 0)),
        ],
        out_specs=pl.BlockSpec(
            (S, D), lambda g: (jnp.where(g < NB, 0, (g - NB) // NHG), 0)),
        scratch_shapes=[
            pltpu.VMEM((B * S, H * Dh), BF16),
            pltpu.VMEM((B * S, H * Dh), BF16),
            pltpu.VMEM((B * S, H * Dh), BF16),
            pltpu.VMEM((B * S, H * Dr), BF16),
            pltpu.VMEM((B * S, Dr), BF16),
            pltpu.VMEM((B * S, DC_SH), BF16),
            pltpu.VMEM((DC_SH, H * Dh), BF16),
            pltpu.VMEM((DC_SH, H * Dh), BF16),
            pltpu.VMEM((B * S, DC_SH), BF16),
            pltpu.VMEM((DC_SH, H * Dh), BF16),
            pltpu.VMEM((DC_SH, H * Dh), BF16),
            pltpu.SemaphoreType.DMA((3,)),
            pltpu.SemaphoreType.DMA((3,)),
        ],
        compiler_params=pltpu.CompilerParams(
            collective_id=0, vmem_limit_bytes=100 * 1024 * 1024),
    )(x2, Wdkv, Wuk, Wuv, Wkr, Wq, Wqr, Wo)

    return out.reshape(B, S, D)


# device time: 69734 ns/iter; 1.0860x vs baseline; 1.0860x over previous
import jax
import jax.numpy as jnp
from jax import lax
from jax.experimental import pallas as pl
from jax.experimental.pallas import tpu as pltpu

B, S, H, Dh, Dr = 2, 512, 16, 128, 32
D = 2048
DC_SH = 128
HG = 4
NB = 4
WQB = D // NB
N_ATTN = B * (H // HG)
F32 = jnp.float32
BF16 = jnp.bfloat16
SCALE = (Dh + Dr) ** -0.5


def _dot(a, b):
    return jnp.dot(a, b, preferred_element_type=F32)


def _mega_body(x_ref, wdkv_ref, wuk_ref, wuv_ref, wkr_ref, wq_ref, wqr_ref,
               o_ref,
               q_s, k_s, v_s, qr_s, kr_s,
               c_ref, wuk_bf, wuv_bf, peer_c, peer_wuk, peer_wuv,
               send_sems, recv_sems):
    g = pl.program_id(0)
    my_x = lax.axis_index("x")
    my_y = lax.axis_index("y")
    my_z = lax.axis_index("z")
    peer = (my_x, 1 - my_y, my_z)

    def exchange_rdmas():
        return [
            pltpu.make_async_remote_copy(
                src_ref=src, dst_ref=dst,
                send_sem=send_sems.at[i], recv_sem=recv_sems.at[i],
                device_id=peer, device_id_type=pl.DeviceIdType.MESH)
            for i, (src, dst) in enumerate(((c_ref, peer_c),
                                            (wuk_bf, peer_wuk),
                                            (wuv_bf, peer_wuv)))
        ]

    @pl.when(g == 0)
    def _():
        barrier_sem = pltpu.get_barrier_semaphore()
        pl.semaphore_signal(barrier_sem, inc=1, device_id=peer,
                            device_id_type=pl.DeviceIdType.MESH)
        pl.semaphore_wait(barrier_sem, 1)

        wuk_bf[...] = wuk_ref[...].astype(BF16)
        wuv_bf[...] = wuv_ref[...].astype(BF16)
        c_ref[...] = _dot(x_ref[...], wdkv_ref[...]).astype(BF16)

        for rdma in exchange_rdmas():
            rdma.start()

        kr_s[...] = _dot(x_ref[...], wkr_ref[...]).astype(BF16)
        qr_s[...] = (_dot(x_ref[...], wqr_ref[...]) * SCALE).astype(BF16)
        k_s[...] = _dot(c_ref[...], wuk_bf[...]).astype(BF16)
        v_s[...] = _dot(c_ref[...], wuv_bf[...]).astype(BF16)

    @pl.when(g < NB)
    def _():
        q_s[:, pl.ds(g * WQB, WQB)] = (
            _dot(x_ref[...], wq_ref[...]) * SCALE).astype(BF16)

    @pl.when(g == NB - 1)
    def _():
        for rdma in exchange_rdmas():
            rdma.wait()
        k_s[...] = (k_s[...] + _dot(peer_c[...], peer_wuk[...])).astype(BF16)
        v_s[...] = (v_s[...] + _dot(peer_c[...], peer_wuv[...])).astype(BF16)

    @pl.when(g >= NB)
    def _():
        j = g - NB
        b = j // (H // HG)
        hg = j % (H // HG)
        row = pl.ds(b * S, S)
        kr = kr_s[row, :]
        qr_grp = qr_s[row, pl.ds(hg * HG * Dr, HG * Dr)]
        contract_last = (((1,), (1,)), ((), ()))
        for i in range(HG):
            col = pl.ds(hg * HG * Dh + i * Dh, Dh)
            q = q_s[row, col]
            qr = qr_grp[:, i * Dr:(i + 1) * Dr]
            k = k_s[row, col]
            v = v_s[row, col]
            scores = (lax.dot_general(q, k, contract_last,
                                      preferred_element_type=F32)
                      + lax.dot_general(qr, kr, contract_last,
                                        preferred_element_type=F32))
            p = jnp.exp(scores)
            recip = 1.0 / jnp.sum(p, axis=-1, keepdims=True)
            o = _dot(p.astype(BF16), v)
            o_ref[row, col] = (o * recip).astype(BF16)


def _out_body(o_ref, wo_ref, out_ref):
    out_ref[...] = _dot(o_ref[...], wo_ref[...])


def kernel(x, Wdkv, Wuk, Wuv, Wq, Wqr, Wkr, Wo):
    x2 = x.reshape(B * S, D)

    o = pl.pallas_call(
        _mega_body,
        grid=(NB + N_ATTN,),
        out_shape=jax.ShapeDtypeStruct((B * S, H * Dh), BF16),
        in_specs=[
            pl.BlockSpec((B * S, D), lambda g: (0, 0)),
            pl.BlockSpec((D, DC_SH), lambda g: (0, 0)),
            pl.BlockSpec((DC_SH, H * Dh), lambda g: (0, 0)),
            pl.BlockSpec((DC_SH, H * Dh), lambda g: (0, 0)),
            pl.BlockSpec((D, Dr), lambda g: (0, 0)),
            pl.BlockSpec((D, WQB),
                         lambda g: (0, jnp.minimum(g, NB - 1))),
            pl.BlockSpec((D, H * Dr), lambda g: (0, 0)),
        ],
        out_specs=pl.BlockSpec((B * S, H * Dh), lambda g: (0, 0)),
        scratch_shapes=[
            pltpu.VMEM((B * S, H * Dh), BF16),
            pltpu.VMEM((B * S, H * Dh), BF16),
            pltpu.VMEM((B * S, H * Dh), BF16),
            pltpu.VMEM((B * S, H * Dr), BF16),
            pltpu.VMEM((B * S, Dr), BF16),
            pltpu.VMEM((B * S, DC_SH), BF16),
            pltpu.VMEM((DC_SH, H * Dh), BF16),
            pltpu.VMEM((DC_SH, H * Dh), BF16),
            pltpu.VMEM((B * S, DC_SH), BF16),
            pltpu.VMEM((DC_SH, H * Dh), BF16),
            pltpu.VMEM((DC_SH, H * Dh), BF16),
            pltpu.SemaphoreType.DMA((3,)),
            pltpu.SemaphoreType.DMA((3,)),
        ],
        compiler_params=pltpu.CompilerParams(
            collective_id=0, vmem_limit_bytes=100 * 1024 * 1024),
    )(x2, Wdkv, Wuk, Wuv, Wkr, Wq, Wqr)

    n_wo_blocks = 4
    wo_blk = D // n_wo_blocks
    out = pl.pallas_call(
        _out_body,
        grid=(n_wo_blocks,),
        out_shape=jax.ShapeDtypeStruct((B * S, D), F32),
        in_specs=[
            pl.BlockSpec((B * S, H * Dh), lambda j: (0, 0)),
            pl.BlockSpec((H * Dh, wo_blk), lambda j: (0, j)),
        ],
        out_specs=pl.BlockSpec((B * S, wo_blk), lambda j: (0, j)),
        compiler_params=pltpu.CompilerParams(
            vmem_limit_bytes=100 * 1024 * 1024),
    )(o, Wo)

    return out.reshape(B, S, D)


# device time: 68310 ns/iter; 1.1086x vs baseline; 1.0208x over previous
import jax
import jax.numpy as jnp
from jax import lax
from jax.experimental import pallas as pl
from jax.experimental.pallas import tpu as pltpu

B, S, H, Dh, Dr = 2, 512, 16, 128, 32
D = 2048
DC_SH = 128
HG = 4
NB = 8
WQB = D // NB
N_ATTN = B * (H // HG)
F32 = jnp.float32
BF16 = jnp.bfloat16
SCALE = (Dh + Dr) ** -0.5


def _dot(a, b):
    return jnp.dot(a, b, preferred_element_type=F32)


def _mega_body(x_ref, wdkv_ref, wuk_ref, wuv_ref, wkr_ref, wq_ref, wqr_ref,
               o_ref,
               q_s, k_s, v_s, qr_s, kr_s,
               c_ref, wuk_bf, wuv_bf, peer_c, peer_wuk, peer_wuv,
               send_sems, recv_sems):
    g = pl.program_id(0)
    my_x = lax.axis_index("x")
    my_y = lax.axis_index("y")
    my_z = lax.axis_index("z")
    peer = (my_x, 1 - my_y, my_z)

    def exchange_rdmas():
        return [
            pltpu.make_async_remote_copy(
                src_ref=src, dst_ref=dst,
                send_sem=send_sems.at[i], recv_sem=recv_sems.at[i],
                device_id=peer, device_id_type=pl.DeviceIdType.MESH)
            for i, (src, dst) in enumerate(((c_ref, peer_c),
                                            (wuk_bf, peer_wuk),
                                            (wuv_bf, peer_wuv)))
        ]

    @pl.when(g == 0)
    def _():
        barrier_sem = pltpu.get_barrier_semaphore()
        pl.semaphore_signal(barrier_sem, inc=1, device_id=peer,
                            device_id_type=pl.DeviceIdType.MESH)
        pl.semaphore_wait(barrier_sem, 1)

        wuk_bf[...] = wuk_ref[...].astype(BF16)
        wuv_bf[...] = wuv_ref[...].astype(BF16)
        c_ref[...] = _dot(x_ref[...], wdkv_ref[...]).astype(BF16)

        for rdma in exchange_rdmas():
            rdma.start()

        kr_s[...] = _dot(x_ref[...], wkr_ref[...]).astype(BF16)
        qr_s[...] = (_dot(x_ref[...], wqr_ref[...]) * SCALE).astype(BF16)
        k_s[...] = _dot(c_ref[...], wuk_bf[...]).astype(BF16)
        v_s[...] = _dot(c_ref[...], wuv_bf[...]).astype(BF16)

    @pl.when(g < NB)
    def _():
        q_s[:, pl.ds(g * WQB, WQB)] = (
            _dot(x_ref[...], wq_ref[...]) * SCALE).astype(BF16)

    @pl.when(g == NB - 1)
    def _():
        for rdma in exchange_rdmas():
            rdma.wait()
        k_s[...] = (k_s[...] + _dot(peer_c[...], peer_wuk[...])).astype(BF16)
        v_s[...] = (v_s[...] + _dot(peer_c[...], peer_wuv[...])).astype(BF16)

    @pl.when(g >= NB)
    def _():
        j = g - NB
        b = j // (H // HG)
        hg = j % (H // HG)
        row = pl.ds(b * S, S)
        kr = kr_s[row, :]
        qr_grp = qr_s[row, pl.ds(hg * HG * Dr, HG * Dr)]
        contract_last = (((1,), (1,)), ((), ()))
        for i in range(HG):
            col = pl.ds(hg * HG * Dh + i * Dh, Dh)
            q = q_s[row, col]
            qr = qr_grp[:, i * Dr:(i + 1) * Dr]
            k = k_s[row, col]
            v = v_s[row, col]
            scores = (lax.dot_general(q, k, contract_last,
                                      preferred_element_type=F32)
                      + lax.dot_general(qr, kr, contract_last,
                                        preferred_element_type=F32))
            p = jnp.exp(scores.astype(BF16))
            recip = 1.0 / jnp.sum(p, axis=-1, keepdims=True, dtype=F32)
            o = _dot(p, v)
            o_ref[row, col] = (o * recip).astype(BF16)


def _out_body(o_ref, wo_ref, out_ref):
    out_ref[...] = _dot(o_ref[...], wo_ref[...])


def kernel(x, Wdkv, Wuk, Wuv, Wq, Wqr, Wkr, Wo):
    x2 = x.reshape(B * S, D)

    o = pl.pallas_call(
        _mega_body,
        grid=(NB + N_ATTN,),
        out_shape=jax.ShapeDtypeStruct((B * S, H * Dh), BF16),
        in_specs=[
            pl.BlockSpec((B * S, D), lambda g: (0, 0)),
            pl.BlockSpec((D, DC_SH), lambda g: (0, 0)),
            pl.BlockSpec((DC_SH, H * Dh), lambda g: (0, 0)),
            pl.BlockSpec((DC_SH, H * Dh), lambda g: (0, 0)),
            pl.BlockSpec((D, Dr), lambda g: (0, 0)),
            pl.BlockSpec((D, WQB),
                         lambda g: (0, jnp.minimum(g, NB - 1))),
            pl.BlockSpec((D, H * Dr), lambda g: (0, 0)),
        ],
        out_specs=pl.BlockSpec((B * S, H * Dh), lambda g: (0, 0)),
        scratch_shapes=[
            pltpu.VMEM((B * S, H * Dh), BF16),
            pltpu.VMEM((B * S, H * Dh), BF16),
            pltpu.VMEM((B * S, H * Dh), BF16),
            pltpu.VMEM((B * S, H * Dr), BF16),
            pltpu.VMEM((B * S, Dr), BF16),
            pltpu.VMEM((B * S, DC_SH), BF16),
            pltpu.VMEM((DC_SH, H * Dh), BF16),
            pltpu.VMEM((DC_SH, H * Dh), BF16),
            pltpu.VMEM((B * S, DC_SH), BF16),
            pltpu.VMEM((DC_SH, H * Dh), BF16),
            pltpu.VMEM((DC_SH, H * Dh), BF16),
            pltpu.SemaphoreType.DMA((3,)),
            pltpu.SemaphoreType.DMA((3,)),
        ],
        compiler_params=pltpu.CompilerParams(
            collective_id=0, vmem_limit_bytes=100 * 1024 * 1024),
    )(x2, Wdkv, Wuk, Wuv, Wkr, Wq, Wqr)

    n_wo_blocks = 4
    wo_blk = D // n_wo_blocks
    out = pl.pallas_call(
        _out_body,
        grid=(n_wo_blocks,),
        out_shape=jax.ShapeDtypeStruct((B * S, D), F32),
        in_specs=[
            pl.BlockSpec((B * S, H * Dh), lambda j: (0, 0)),
            pl.BlockSpec((H * Dh, wo_blk), lambda j: (0, j)),
        ],
        out_specs=pl.BlockSpec((B * S, wo_blk), lambda j: (0, j)),
        compiler_params=pltpu.CompilerParams(
            vmem_limit_bytes=100 * 1024 * 1024),
    )(o, Wo)

    return out.reshape(B, S, D)
